# baseline (speedup 1.0000x reference)
import functools

import jax
import jax.numpy as jnp
from jax import lax
from jax.experimental import pallas as pl
from jax.experimental.pallas import tpu as pltpu
from jax.experimental.pallas import tpu_sc as plsc

_NC = 2
_NS = 16
_NW = _NC * _NS

_P = 200
_NT = 16384
_D = 32
_IC = _NT // 128
_UPT = _P * _IC // _NW


@functools.lru_cache(maxsize=None)
def _build():
    mesh = plsc.VectorSubcoreMesh(core_axis_name="c", subcore_axis_name="s")

    @functools.partial(
        pl.kernel,
        mesh=mesh,
        compiler_params=pltpu.CompilerParams(
            use_tc_tiling_on_sc=False, needs_layout_passes=False
        ),
        out_type=jax.ShapeDtypeStruct((_P, _D // 8, _IC, 8, 128), jnp.float32),
        scratch_types=[
            pltpu.VMEM((128,), jnp.int32),
            pltpu.VMEM((128,), jnp.int32),
            pltpu.VMEM((128, _D), jnp.float32),
            pltpu.VMEM((128, _D), jnp.float32),
            pltpu.VMEM((_D, 128), jnp.float32),
            pltpu.VMEM((_D, 128), jnp.float32),
            pltpu.SemaphoreType.DMA,
            pltpu.SemaphoreType.DMA,
            pltpu.SemaphoreType.DMA,
            pltpu.SemaphoreType.DMA,
        ],
    )
    def k(x4_hbm, lut_hbm, out_hbm, i0, i1, r0, r1, t0, t1, g0, g1, o0, o1):
        idx_v = (i0, i1)
        rows_v = (r0, r1)
        tile_v = (t0, t1)
        gsem = (g0, g1)
        osem = (o0, o1)
        wid = lax.axis_index("s") * _NC + lax.axis_index("c")
        u0 = wid * _UPT

        toks = [
            jnp.arange(16, dtype=jnp.int32) + t8 * 16 for t8 in range(8)
        ]

        def unit_coords(u):
            j = u // _IC
            ic = u % _IC
            return j, ic, j // 8, j % 8

        def load_idx(u, b):
            _, ic, jr, j8 = unit_coords(u)
            pltpu.sync_copy(x4_hbm.at[jr, ic, j8], idx_v[b])

        def gather(b):
            return pltpu.make_async_copy(
                lut_hbm.at[idx_v[b]], rows_v[b], gsem[b]
            )

        def out_copies(u, b):
            j, ic, _, _ = unit_coords(u)
            return [
                pltpu.make_async_copy(
                    tile_v[b].at[pl.ds(fr * 8, 8)],
                    out_hbm.at[j, fr, ic],
                    osem[b],
                )
                for fr in range(4)
            ]

        load_idx(u0, 0)
        gather(0).start()

        @pl.loop(0, _UPT, step=2)
        def _outer(n0):
            for b in range(2):
                n = n0 + b
                u = u0 + n

                gather(b).wait()

                @pl.when(n + 1 < _UPT)
                def _():
                    load_idx(u0 + n + 1, 1 - b)
                    gather(1 - b).start()

                @pl.when(n >= 2)
                def _():
                    for c in out_copies(u - 2, b):
                        c.wait()

                @pl.loop(0, _D)
                def _row(f):
                    fv = jnp.full((16,), f, dtype=jnp.int32)
                    for t8 in range(8):
                        vals = plsc.load_gather(rows_v[b], [toks[t8], fv])
                        tile_v[b][f, pl.ds(t8 * 16, 16)] = vals

                for c in out_copies(u, b):
                    c.start()

        for n in (_UPT - 2, _UPT - 1):
            for c in out_copies(u0 + n, n % 2):
                c.wait()

    return k


def kernel(x, lut):
    x4 = x.T.reshape(_P // 8, 8, _IC, 128).transpose(0, 2, 1, 3)
    out5 = _build()(x4, lut)
    out = out5.transpose(2, 4, 0, 1, 3).reshape(_NT, _P, _D)
    return out
